# Initial kernel scaffold; baseline (speedup 1.0000x reference)
import jax
import jax.numpy as jnp
from jax.experimental import pallas as pl


def kernel(indices, table):
    raise NotImplementedError("write your pallas kernel here")


# speedup vs baseline: 1.2875x; 1.2875x over previous
import functools

import jax
import jax.numpy as jnp
from jax import lax
from jax.experimental import pallas as pl
from jax.experimental.pallas import tpu as pltpu
from jax.experimental.pallas import tpu_sc as plsc

_NC = 2
_NS = 16
_NW = _NC * _NS
_C = 128
_NBUF = 5


@functools.partial(jax.jit, static_argnames=("cpw", "dim"))
def _gather_sc(idx3, table, cpw, dim):
    n = _NW * cpw * _C
    assert cpw % _NBUF == 0 and cpw >= 2 * _NBUF
    mesh = plsc.VectorSubcoreMesh(core_axis_name="c", subcore_axis_name="s")

    @functools.partial(
        pl.kernel,
        out_type=jax.ShapeDtypeStruct((n, dim), table.dtype),
        mesh=mesh,
        scratch_types=[
            pltpu.VMEM((cpw, _C), jnp.int32),
            pltpu.VMEM((_NBUF, _C, dim), table.dtype),
            pltpu.SemaphoreType.DMA((_NBUF,)),
            pltpu.SemaphoreType.DMA((_NBUF,)),
        ],
    )
    def k(idx_hbm, table_hbm, out_hbm, idx_v, rows_v, gsem, wsem):
        wid = lax.axis_index("s") * _NC + lax.axis_index("c")
        pltpu.sync_copy(idx_hbm.at[wid], idx_v)
        base = wid * (cpw * _C)

        def fire_gather(j, b):
            return pltpu.async_copy(
                table_hbm.at[idx_v.at[j]], rows_v.at[b], gsem.at[b])

        def wait_gather(j, b):
            pltpu.make_async_copy(
                table_hbm.at[idx_v.at[j]], rows_v.at[b], gsem.at[b]).wait()

        def fire_write(j, b):
            return pltpu.async_copy(
                rows_v.at[b], out_hbm.at[pl.ds(base + j * _C, _C)], wsem.at[b])

        def wait_write(j, b):
            pltpu.make_async_copy(
                rows_v.at[b], out_hbm.at[pl.ds(base + j * _C, _C)],
                wsem.at[b]).wait()

        for b in range(_NBUF):
            fire_gather(b, b)

        @pl.loop(0, cpw - _NBUF, step=_NBUF)
        def _(j0):
            for b in range(_NBUF):
                j = j0 + b
                wait_gather(j, b)
                fire_write(j, b)
                wait_write(j, b)
                fire_gather(j + _NBUF, b)

        for b in range(_NBUF):
            j = cpw - _NBUF + b
            wait_gather(j, b)
            fire_write(j, b)
        for b in range(_NBUF):
            wait_write(cpw - _NBUF + b, b)

    return k(idx3, table)


def kernel(indices, table):
    b, l = indices.shape
    dim = table.shape[1]
    n = b * l
    assert n % (_NW * _C) == 0
    cpw = n // (_NW * _C)
    idx3 = indices.reshape(_NW, cpw, _C)
    out = _gather_sc(idx3, table, cpw, dim)
    return out.reshape(b, l, dim)


# speedup vs baseline: 4.0290x; 3.1292x over previous
import functools

import jax
import jax.numpy as jnp
from jax import lax
from jax.experimental import pallas as pl
from jax.experimental.pallas import tpu as pltpu
from jax.experimental.pallas import tpu_sc as plsc

_NC = 2
_NS = 16
_NW = _NC * _NS
_C = 128
_NBUF = 5


@functools.partial(jax.jit, static_argnames=("cpw", "dim"))
def _gather_sc(idx3, table, cpw, dim):
    n = _NW * cpw * _C
    assert cpw % _NBUF == 0 and cpw >= 2 * _NBUF
    mesh = plsc.VectorSubcoreMesh(core_axis_name="c", subcore_axis_name="s")

    @functools.partial(
        pl.kernel,
        out_type=jax.ShapeDtypeStruct((n, dim), table.dtype),
        mesh=mesh,
        scratch_types=[
            pltpu.VMEM((cpw, _C), jnp.int32),
            pltpu.VMEM((_NBUF, _C, dim), table.dtype),
            pltpu.SemaphoreType.DMA((_NBUF,)),
            pltpu.SemaphoreType.DMA((_NBUF,)),
        ],
    )
    def k(idx_hbm, table_hbm, out_hbm, idx_v, rows_v, gsem, wsem):
        wid = lax.axis_index("s") * _NC + lax.axis_index("c")
        pltpu.sync_copy(idx_hbm.at[wid], idx_v)
        base = wid * (cpw * _C)

        def fire_gather(j, b):
            return pltpu.async_copy(
                table_hbm.at[idx_v.at[j]], rows_v.at[b], gsem.at[b])

        def wait_gather(j, b):
            pltpu.make_async_copy(
                table_hbm.at[idx_v.at[j]], rows_v.at[b], gsem.at[b]).wait()

        def fire_write(j, b):
            return pltpu.async_copy(
                rows_v.at[b], out_hbm.at[pl.ds(base + j * _C, _C)], wsem.at[b])

        def wait_write(j, b):
            pltpu.make_async_copy(
                rows_v.at[b], out_hbm.at[pl.ds(base + j * _C, _C)],
                wsem.at[b]).wait()

        for b in range(_NBUF):
            fire_gather(b, b)

        @pl.loop(0, cpw - _NBUF, step=_NBUF)
        def _(j0):
            for b in range(_NBUF):
                j = j0 + b
                wait_gather(j, b)
                fire_write(j, b)
                wait_write(j, b)
                fire_gather(j + _NBUF, b)

        for b in range(_NBUF):
            j = cpw - _NBUF + b
            wait_gather(j, b)
            fire_write(j, b)
        for b in range(_NBUF):
            wait_write(cpw - _NBUF + b, b)

    return k(idx3, table)


def kernel(indices, table):
    b, l = indices.shape
    dim = table.shape[1]
    n = b * l
    assert n % (_NW * _C) == 0
    cpw = n // (_NW * _C)
    idx3 = indices.T.reshape(_NW, cpw, _C)
    out = _gather_sc(idx3, table, cpw, dim)
    return out.reshape(l, b, dim).transpose(1, 0, 2)
